# baseline (speedup 1.0000x reference)
import functools

import jax
import jax.numpy as jnp
from jax import lax
from jax.experimental import pallas as pl
from jax.experimental.pallas import tpu as pltpu
from jax.experimental.pallas import tpu_sc as plsc

NUM_EMBEDDINGS = 1000000
EMBEDDING_DIM = 64
BATCH = 16384


def _make_lookup():
    info = plsc.get_sparse_core_info()
    nw = info.num_cores * info.num_subcores
    b_per_w = BATCH // nw

    mesh = plsc.VectorSubcoreMesh(core_axis_name="c", subcore_axis_name="s")

    @functools.partial(
        pl.kernel,
        mesh=mesh,
        out_type=jax.ShapeDtypeStruct((BATCH, EMBEDDING_DIM), jnp.float32),
        scratch_types=[
            pltpu.VMEM((b_per_w,), jnp.int32),
            pltpu.VMEM((b_per_w, EMBEDDING_DIM), jnp.float32),
            pltpu.SemaphoreType.DMA,
        ],
        compiler_params=pltpu.CompilerParams(use_tc_tiling_on_sc=False),
    )
    def lookup(idx_hbm, table_hbm, out_hbm, idx_v, rows_v, sem):
        wid = lax.axis_index("s") * info.num_cores + lax.axis_index("c")
        base = wid * b_per_w
        pltpu.sync_copy(idx_hbm.at[pl.ds(base, b_per_w)], idx_v)
        pltpu.async_copy(table_hbm.at[idx_v], rows_v, sem).wait()
        pltpu.sync_copy(rows_v, out_hbm.at[pl.ds(base, b_per_w)])

    return lookup


_lookup = _make_lookup()


def kernel(X, table):
    return _lookup(X.astype(jnp.int32), table)


# speedup vs baseline: 3.0171x; 3.0171x over previous
import functools

import jax
import jax.numpy as jnp
from jax import lax
from jax.experimental import pallas as pl
from jax.experimental.pallas import tpu as pltpu
from jax.experimental.pallas import tpu_sc as plsc

NUM_EMBEDDINGS = 1000000
EMBEDDING_DIM = 64
BATCH = 16384
NBUF = 8


def _make_lookup():
    info = plsc.get_sparse_core_info()
    nw = info.num_cores * info.num_subcores
    b_per_w = BATCH // nw

    mesh = plsc.VectorSubcoreMesh(core_axis_name="c", subcore_axis_name="s")

    @functools.partial(
        pl.kernel,
        mesh=mesh,
        out_type=jax.ShapeDtypeStruct((EMBEDDING_DIM, BATCH), jnp.float32),
        scratch_types=[
            pltpu.VMEM((b_per_w,), jnp.int32),
            pltpu.VMEM((b_per_w,), jnp.int32),
            pltpu.VMEM((NBUF, EMBEDDING_DIM, 128), jnp.float32),
            pltpu.VMEM((EMBEDDING_DIM, b_per_w), jnp.float32),
        ]
        + [pltpu.SemaphoreType.DMA] * NBUF,
        compiler_params=pltpu.CompilerParams(needs_layout_passes=False),
    )
    def lookup(x_hbm, tt_hbm, ot_hbm, c_v, l_v, bufs, stage, *sems):
        wid = lax.axis_index("s") * info.num_cores + lax.axis_index("c")
        base = wid * b_per_w
        pltpu.sync_copy(x_hbm.at[pl.ds(base, b_per_w)], c_v)

        def split_idx(k, _):
            v = c_v[pl.ds(k * 16, 16)]
            l_v[pl.ds(k * 16, 16)] = lax.bitwise_and(v, 127)
            c_v[pl.ds(k * 16, 16)] = lax.shift_right_logical(v, 7)
            return 0

        lax.fori_loop(0, b_per_w // 16, split_idx, 0)

        dvecs = [lax.iota(jnp.int32, 16) + 16 * q for q in range(4)]

        def start_fetch(slot, c_scalar):
            col0 = pl.multiple_of(c_scalar * 128, 128)
            pltpu.async_copy(
                tt_hbm.at[:, pl.ds(col0, 128)], bufs.at[slot], sems[slot]
            )

        def wait_fetch(slot):
            pltpu.make_async_copy(
                tt_hbm.at[:, pl.ds(0, 128)], bufs.at[slot], sems[slot]
            ).wait()

        def extract(slot, lane, n):
            nvec = jnp.full((16,), n, dtype=jnp.int32)
            lvec = jnp.full((16,), lane, dtype=jnp.int32)
            for q in range(4):
                vals = plsc.load_gather(bufs.at[slot], [dvecs[q], lvec])
                plsc.store_scatter(stage, [dvecs[q], nvec], vals)

        cvec0 = c_v[pl.ds(0, 16)]
        for j in range(NBUF):
            start_fetch(j, cvec0[j])

        def per_group(g, _):
            i0 = g * 16
            cvec_lo = c_v[pl.ds(i0, 16)]
            cvec_hi = c_v[pl.ds(i0 + 16, 16)]
            lvec_lo = l_v[pl.ds(i0, 16)]
            for j in range(16):
                n = i0 + j
                slot = j % NBUF
                wait_fetch(slot)
                extract(slot, lvec_lo[j], n)
                m_lane = j + NBUF
                if m_lane < 16:
                    start_fetch(slot, cvec_lo[m_lane])
                else:
                    start_fetch(slot, cvec_hi[m_lane - 16])
            return 0

        lax.fori_loop(0, b_per_w // 16 - 1, per_group, 0)

        i0 = b_per_w - 16
        cvec_lo = c_v[pl.ds(i0, 16)]
        lvec_lo = l_v[pl.ds(i0, 16)]
        for j in range(16):
            n = i0 + j
            slot = j % NBUF
            wait_fetch(slot)
            extract(slot, lvec_lo[j], n)
            if j + NBUF < 16:
                start_fetch(slot, cvec_lo[j + NBUF])

        pltpu.sync_copy(stage, ot_hbm.at[:, pl.ds(base, b_per_w)])

    return lookup


_lookup = _make_lookup()


def kernel(X, table):
    out_t = _lookup(X.astype(jnp.int32), table.T)
    return out_t.T
